# baseline (speedup 1.0000x reference)
import jax
import jax.numpy as jnp
from jax import lax
from jax.experimental import pallas as pl
from jax.experimental.pallas import tpu as pltpu
from jax.experimental.pallas import tpu_sc as plsc

_NC = 2
_NS = 16
_NW = _NC * _NS
_L = 16
_CHUNK = 256


def _sc_body(gum_hbm, gim_hbm, out_hbm, a_v, b_v, o_v):
    rows = o_v.shape[0]
    chunk = a_v.shape[0]
    n_cols = a_v.shape[1]
    wid = lax.axis_index("s") * _NC + lax.axis_index("c")
    base = wid * rows

    lanes = lax.iota(jnp.int32, _L)

    for h in range(rows // chunk):
        pltpu.sync_copy(gum_hbm.at[pl.ds(base + h * chunk, chunk), :], a_v)
        pltpu.sync_copy(gim_hbm.at[pl.ds(base + h * chunk, chunk), :], b_v)

        def group(g, carry):
            row_idx = g * _L + lanes
            acc = jnp.zeros((_L,), jnp.float32)
            for c in range(n_cols):
                col_idx = jnp.bitwise_and(c + lanes, n_cols - 1)
                ga = plsc.load_gather(a_v, [row_idx, col_idx])
                gb = plsc.load_gather(b_v, [row_idx, col_idx])
                acc = acc + ga * gb
            o_v[pl.ds(h * chunk + g * _L, _L)] = acc
            return carry

        lax.fori_loop(0, chunk // _L, group, 0)

    pltpu.sync_copy(o_v, out_hbm.at[pl.ds(base, rows)])


def kernel(gum, gim):
    n_rows, n_cols = gum.shape
    rows_per_w = n_rows // _NW
    mesh = plsc.VectorSubcoreMesh(core_axis_name="c", subcore_axis_name="s")
    xui = pl.kernel(
        _sc_body,
        out_type=jax.ShapeDtypeStruct((n_rows,), jnp.float32),
        mesh=mesh,
        compiler_params=pltpu.CompilerParams(
            needs_layout_passes=False, use_tc_tiling_on_sc=True
        ),
        scratch_types=[
            pltpu.VMEM((_CHUNK, n_cols), jnp.float32),
            pltpu.VMEM((_CHUNK, n_cols), jnp.float32),
            pltpu.VMEM((rows_per_w,), jnp.float32),
        ],
    )(gum, gim)
    return (xui, gum, gim)


# speedup vs baseline: 1.1995x; 1.1995x over previous
import jax
import jax.numpy as jnp
from jax.experimental import pallas as pl

_BLOCK = 2048


def _body(gum_ref, gim_ref, xui_ref, gu_out_ref, gi_out_ref):
    gu = gum_ref[...]
    gi = gim_ref[...]
    gu_out_ref[...] = gu
    gi_out_ref[...] = gi
    xui_ref[...] = jnp.sum(gu * gi, axis=1)


def kernel(gum, gim):
    n_rows, n_cols = gum.shape
    grid = (n_rows // _BLOCK,)
    xui, gu_o, gi_o = pl.pallas_call(
        _body,
        grid=grid,
        in_specs=[
            pl.BlockSpec((_BLOCK, n_cols), lambda i: (i, 0)),
            pl.BlockSpec((_BLOCK, n_cols), lambda i: (i, 0)),
        ],
        out_specs=[
            pl.BlockSpec((_BLOCK,), lambda i: (i,)),
            pl.BlockSpec((_BLOCK, n_cols), lambda i: (i, 0)),
            pl.BlockSpec((_BLOCK, n_cols), lambda i: (i, 0)),
        ],
        out_shape=[
            jax.ShapeDtypeStruct((n_rows,), jnp.float32),
            jax.ShapeDtypeStruct((n_rows, n_cols), jnp.float32),
            jax.ShapeDtypeStruct((n_rows, n_cols), jnp.float32),
        ],
    )(gum, gim)
    return (xui, gu_o, gi_o)


# speedup vs baseline: 5.8612x; 4.8864x over previous
import jax
import jax.numpy as jnp
from jax.experimental import pallas as pl

_BN = 4096


def _body(a_ref, b_ref, xui_ref, a_out_ref, b_out_ref):
    av = a_ref[...]
    bv = b_ref[...]
    a_out_ref[...] = av
    b_out_ref[...] = bv
    xui_ref[...] = jnp.sum(av * bv, axis=0)


def kernel(gum, gim):
    n_rows, n_cols = gum.shape
    a = gum.T
    b = gim.T
    grid = (n_rows // _BN,)
    xui, a_o, b_o = pl.pallas_call(
        _body,
        grid=grid,
        in_specs=[
            pl.BlockSpec((n_cols, _BN), lambda i: (0, i)),
            pl.BlockSpec((n_cols, _BN), lambda i: (0, i)),
        ],
        out_specs=[
            pl.BlockSpec((_BN,), lambda i: (i,)),
            pl.BlockSpec((n_cols, _BN), lambda i: (0, i)),
            pl.BlockSpec((n_cols, _BN), lambda i: (0, i)),
        ],
        out_shape=[
            jax.ShapeDtypeStruct((n_rows,), jnp.float32),
            jax.ShapeDtypeStruct((n_cols, n_rows), jnp.float32),
            jax.ShapeDtypeStruct((n_cols, n_rows), jnp.float32),
        ],
    )(a, b)
    return (xui, a_o.T, b_o.T)


# speedup vs baseline: 7.2581x; 1.2383x over previous
import jax
import jax.numpy as jnp
from jax.experimental import pallas as pl

_BN = 8192


def _body(a_ref, b_ref, xui_ref, a_out_ref, b_out_ref):
    av = a_ref[...]
    bv = b_ref[...]
    a_out_ref[...] = av
    b_out_ref[...] = bv
    xui_ref[...] = jnp.sum(av * bv, axis=0)


def kernel(gum, gim):
    n_rows, n_cols = gum.shape
    a = gum.T
    b = gim.T
    grid = (n_rows // _BN,)
    xui, a_o, b_o = pl.pallas_call(
        _body,
        grid=grid,
        in_specs=[
            pl.BlockSpec((n_cols, _BN), lambda i: (0, i)),
            pl.BlockSpec((n_cols, _BN), lambda i: (0, i)),
        ],
        out_specs=[
            pl.BlockSpec((_BN,), lambda i: (i,)),
            pl.BlockSpec((n_cols, _BN), lambda i: (0, i)),
            pl.BlockSpec((n_cols, _BN), lambda i: (0, i)),
        ],
        out_shape=[
            jax.ShapeDtypeStruct((n_rows,), jnp.float32),
            jax.ShapeDtypeStruct((n_cols, n_rows), jnp.float32),
            jax.ShapeDtypeStruct((n_cols, n_rows), jnp.float32),
        ],
    )(a, b)
    return (xui, a_o.T, b_o.T)
